# Initial kernel scaffold; baseline (speedup 1.0000x reference)
import functools

import jax
import jax.numpy as jnp
from jax import lax
from jax.experimental import pallas as pl
from jax.experimental.pallas import tpu as pltpu
from jax.experimental.pallas import tpu_sc as plsc

BATCH = 4096
NUM_FIELDS = 26
NC = 2
NS = 16
LANES = 16
NW = NC * NS
ROWS_PER_W = BATCH // NW
IDX_PER_W = ROWS_PER_W * NUM_FIELDS
GATHER_W = 128
NGATHER = IDX_PER_W // GATHER_W


def _sc_kernel(x_hbm, table_hbm, bias_hbm, out_hbm, idx_v, vals_v, out_v,
               bias_v, sem):
    wid = lax.axis_index("s") * NC + lax.axis_index("c")

    pltpu.sync_copy(x_hbm.at[wid], idx_v)
    pltpu.sync_copy(bias_hbm, bias_v)

    copies = []
    for c in range(NGATHER):
        copies.append(
            pltpu.async_copy(
                table_hbm.at[idx_v.at[c]],
                vals_v.at[c],
                sem,
            ))
    for cp in copies:
        cp.wait()

    bias_vec = bias_v[...]

    def chunk_body(j, _):
        rows = lax.iota(jnp.int32, LANES) + j * LANES
        base_pos = rows * NUM_FIELDS
        acc = bias_vec
        for f in range(NUM_FIELDS):
            pos = base_pos + f
            acc = acc + plsc.load_gather(vals_v, [pos // GATHER_W,
                                                  pos % GATHER_W])
        out_v[pl.ds(j * LANES, LANES)] = acc
        return 0

    lax.fori_loop(0, ROWS_PER_W // LANES, chunk_body, 0, unroll=True)

    pltpu.sync_copy(out_v, out_hbm.at[pl.ds(wid * ROWS_PER_W, ROWS_PER_W)])


@jax.jit
def _run(x_r, table_flat, bias16):
    mesh = plsc.VectorSubcoreMesh(
        core_axis_name="c", subcore_axis_name="s",
        num_cores=NC, num_subcores=NS)
    f = functools.partial(
        pl.kernel,
        out_type=jax.ShapeDtypeStruct((BATCH,), jnp.float32),
        mesh=mesh,
        scratch_types=[
            pltpu.VMEM((NGATHER, GATHER_W), jnp.int32),
            pltpu.VMEM((NGATHER, GATHER_W), jnp.float32),
            pltpu.VMEM((ROWS_PER_W,), jnp.float32),
            pltpu.VMEM((LANES,), jnp.float32),
            pltpu.SemaphoreType.DMA,
        ],
    )(_sc_kernel)
    return f(x_r, table_flat, bias16)


def kernel(x, table, bias):
    x_r = x.astype(jnp.int32).reshape(NW, NGATHER, GATHER_W)
    table_flat = table.reshape(-1)
    bias16 = jnp.broadcast_to(bias.astype(jnp.float32), (LANES,))
    out = _run(x_r, table_flat, bias16)
    return out.reshape(BATCH, 1)


# speedup vs baseline: 1.1149x; 1.1149x over previous
import functools

import jax
import jax.numpy as jnp
from jax import lax
from jax.experimental import pallas as pl
from jax.experimental.pallas import tpu as pltpu
from jax.experimental.pallas import tpu_sc as plsc

BATCH = 4096
NUM_FIELDS = 26
NC = 2
NS = 16
LANES = 16
NW = NC * NS
ROWS_PER_W = BATCH // NW
IDX_PER_W = ROWS_PER_W * NUM_FIELDS
GATHER_W = 128
NGATHER = IDX_PER_W // GATHER_W


def _sc_kernel(x_hbm, table_hbm, bias_hbm, out_hbm, idx_v, vals_v, out_v,
               bias_v, sem):
    wid = lax.axis_index("s") * NC + lax.axis_index("c")

    pltpu.sync_copy(x_hbm.at[wid], idx_v)
    pltpu.sync_copy(bias_hbm, bias_v)

    copies = []
    for c in range(NGATHER):
        copies.append(
            pltpu.async_copy(
                table_hbm.at[idx_v.at[c]],
                vals_v.at[pl.ds(c * GATHER_W, GATHER_W)],
                sem,
            ))
    for cp in copies:
        cp.wait()

    bias_vec = bias_v[...]

    for j in range(ROWS_PER_W // LANES):
        rows = lax.iota(jnp.int32, LANES) + j * LANES
        base_pos = rows * NUM_FIELDS
        acc = bias_vec
        for f in range(NUM_FIELDS):
            acc = acc + plsc.load_gather(vals_v, [base_pos + f])
        out_v[pl.ds(j * LANES, LANES)] = acc

    pltpu.sync_copy(out_v, out_hbm.at[pl.ds(wid * ROWS_PER_W, ROWS_PER_W)])


@jax.jit
def _run(x_r, table_flat, bias16):
    mesh = plsc.VectorSubcoreMesh(
        core_axis_name="c", subcore_axis_name="s",
        num_cores=NC, num_subcores=NS)
    f = functools.partial(
        pl.kernel,
        out_type=jax.ShapeDtypeStruct((BATCH,), jnp.float32),
        mesh=mesh,
        scratch_types=[
            pltpu.VMEM((NGATHER, GATHER_W), jnp.int32),
            pltpu.VMEM((IDX_PER_W,), jnp.float32),
            pltpu.VMEM((ROWS_PER_W,), jnp.float32),
            pltpu.VMEM((LANES,), jnp.float32),
            pltpu.SemaphoreType.DMA,
        ],
        compiler_params=pltpu.CompilerParams(needs_layout_passes=False),
    )(_sc_kernel)
    return f(x_r, table_flat, bias16)


def kernel(x, table, bias):
    x_r = x.astype(jnp.int32).reshape(NW, NGATHER, GATHER_W)
    table_flat = table.reshape(-1)
    bias16 = jnp.broadcast_to(bias.astype(jnp.float32), (LANES,))
    out = _run(x_r, table_flat, bias16)
    return out.reshape(BATCH, 1)


# speedup vs baseline: 1.3146x; 1.1792x over previous
import functools

import jax
import jax.numpy as jnp
from jax import lax
from jax.experimental import pallas as pl
from jax.experimental.pallas import tpu as pltpu
from jax.experimental.pallas import tpu_sc as plsc

BATCH = 4096
NUM_FIELDS = 26
NC = 2
NS = 16
LANES = 16
NW = NC * NS
ROWS_PER_W = BATCH // NW


def _sc_kernel(xt_hbm, table_hbm, bias_hbm, out_hbm, idx_v, vals_v, out_v,
               bias_v, sem):
    wid = lax.axis_index("s") * NC + lax.axis_index("c")
    base = wid * ROWS_PER_W

    pltpu.sync_copy(xt_hbm.at[:, pl.ds(base, ROWS_PER_W)], idx_v)
    pltpu.sync_copy(bias_hbm, bias_v)

    copies = []
    for f in range(NUM_FIELDS):
        copies.append(
            pltpu.async_copy(
                table_hbm.at[idx_v.at[f]],
                vals_v.at[f],
                sem,
            ))
    for cp in copies:
        cp.wait()

    bias_vec = plsc.load_gather(bias_v, [jnp.zeros((LANES,), jnp.int32)])

    for j in range(ROWS_PER_W // LANES):
        acc = bias_vec
        for f in range(NUM_FIELDS):
            acc = acc + vals_v[f, pl.ds(j * LANES, LANES)]
        out_v[pl.ds(j * LANES, LANES)] = acc

    pltpu.sync_copy(out_v, out_hbm.at[pl.ds(base, ROWS_PER_W)])


@jax.jit
def _run(xt, table_flat, bias):
    mesh = plsc.VectorSubcoreMesh(
        core_axis_name="c", subcore_axis_name="s",
        num_cores=NC, num_subcores=NS)
    f = functools.partial(
        pl.kernel,
        out_type=jax.ShapeDtypeStruct((BATCH,), jnp.float32),
        mesh=mesh,
        scratch_types=[
            pltpu.VMEM((NUM_FIELDS, ROWS_PER_W), jnp.int32),
            pltpu.VMEM((NUM_FIELDS, ROWS_PER_W), jnp.float32),
            pltpu.VMEM((ROWS_PER_W,), jnp.float32),
            pltpu.VMEM((1,), jnp.float32),
            pltpu.SemaphoreType.DMA,
        ],
        compiler_params=pltpu.CompilerParams(needs_layout_passes=False),
    )(_sc_kernel)
    return f(xt, table_flat, bias)


def kernel(x, table, bias):
    xt = x.astype(jnp.int32).T
    table_flat = table.reshape(-1)
    out = _run(xt, table_flat, bias.astype(jnp.float32))
    return out.reshape(BATCH, 1)


# speedup vs baseline: 1.3382x; 1.0180x over previous
import functools

import jax
import jax.numpy as jnp
from jax import lax
from jax.experimental import pallas as pl
from jax.experimental.pallas import tpu as pltpu
from jax.experimental.pallas import tpu_sc as plsc

BATCH = 4096
NUM_FIELDS = 26
NC = 2
NS = 16
LANES = 16
NW = NC * NS
ROWS_PER_W = BATCH // NW


def _sc_kernel(xt_hbm, table_hbm, bias_hbm, out_hbm, idx_v, vals_v, out_v,
               bias_v, sem):
    wid = lax.axis_index("s") * NC + lax.axis_index("c")
    base = wid * ROWS_PER_W

    stage = [
        pltpu.async_copy(
            xt_hbm.at[f, pl.ds(base, ROWS_PER_W)],
            idx_v.at[pl.ds(f * ROWS_PER_W, ROWS_PER_W)],
            sem,
        )
        for f in range(NUM_FIELDS)
    ]
    pltpu.sync_copy(bias_hbm, bias_v)
    for cp in stage:
        cp.wait()

    pltpu.async_copy(table_hbm.at[idx_v], vals_v, sem).wait()

    bias_vec = plsc.load_gather(bias_v, [jnp.zeros((LANES,), jnp.int32)])

    for j in range(ROWS_PER_W // LANES):
        acc = bias_vec
        for f in range(NUM_FIELDS):
            acc = acc + vals_v[pl.ds(f * ROWS_PER_W + j * LANES, LANES)]
        out_v[pl.ds(j * LANES, LANES)] = acc

    pltpu.sync_copy(out_v, out_hbm.at[pl.ds(base, ROWS_PER_W)])


@jax.jit
def _run(xt, table_flat, bias):
    mesh = plsc.VectorSubcoreMesh(
        core_axis_name="c", subcore_axis_name="s",
        num_cores=NC, num_subcores=NS)
    f = functools.partial(
        pl.kernel,
        out_type=jax.ShapeDtypeStruct((BATCH,), jnp.float32),
        mesh=mesh,
        scratch_types=[
            pltpu.VMEM((NUM_FIELDS * ROWS_PER_W,), jnp.int32),
            pltpu.VMEM((NUM_FIELDS * ROWS_PER_W,), jnp.float32),
            pltpu.VMEM((ROWS_PER_W,), jnp.float32),
            pltpu.VMEM((1,), jnp.float32),
            pltpu.SemaphoreType.DMA,
        ],
        compiler_params=pltpu.CompilerParams(needs_layout_passes=False),
    )(_sc_kernel)
    return f(xt, table_flat, bias)


def kernel(x, table, bias):
    xt = x.astype(jnp.int32).T
    table_flat = table.reshape(-1)
    out = _run(xt, table_flat, bias.astype(jnp.float32))
    return out.reshape(BATCH, 1)
